# baseline (speedup 1.0000x reference)
import functools

import jax
import jax.numpy as jnp
from jax import lax
from jax.experimental import pallas as pl
from jax.experimental.pallas import tpu as pltpu
from jax.experimental.pallas import tpu_sc as plsc

_VOCAB = 1000
_HIDDEN = 16
_BATCH = 4096
_SEQ = 20
_NB = _BATCH * _SEQ

_NC = 2
_NS = 16
_NW = _NC * _NS
_B_PER_W = _NB // _NW
_CHUNK = 40
_NCHUNK = _B_PER_W // _CHUNK


def _fuse_table_kernel(e_ref, w_ref, b_ref, out_ref):
    m = lax.dot_general(
        e_ref[...], w_ref[...],
        dimension_numbers=(((1,), (1,)), ((), ())),
        preferred_element_type=jnp.float32,
    )
    out_ref[...] = m + b_ref[...]


def _build_fused_table(embed_table, fc_w, fc_b):
    return pl.pallas_call(
        _fuse_table_kernel,
        out_shape=jax.ShapeDtypeStruct((_VOCAB, _VOCAB), jnp.float32),
    )(embed_table, fc_w, fc_b.reshape(1, _VOCAB))


@functools.cache
def _make_gather_rows():
    mesh = plsc.VectorSubcoreMesh(core_axis_name="c", subcore_axis_name="s")

    @functools.partial(
        pl.kernel,
        mesh=mesh,
        out_type=jax.ShapeDtypeStruct((_NB, _VOCAB), jnp.float32),
        scratch_types=[
            pltpu.VMEM((_B_PER_W,), jnp.int32),
            pltpu.VMEM((_CHUNK, _VOCAB), jnp.float32),
            pltpu.SemaphoreType.DMA,
        ],
        compiler_params=pltpu.CompilerParams(use_tc_tiling_on_sc=False),
    )
    def _gather_rows(m_hbm, idx_hbm, out_hbm, idx_v, rows_v, sem):
        wid = lax.axis_index("s") * _NC + lax.axis_index("c")
        base = pl.multiple_of(wid * _B_PER_W, 8)
        pltpu.sync_copy(idx_hbm.at[pl.ds(base, _B_PER_W)], idx_v)

        def body(c, carry):
            off = pl.multiple_of(c * _CHUNK, 8)
            pltpu.async_copy(
                m_hbm.at[idx_v.at[pl.ds(off, _CHUNK)]], rows_v, sem
            ).wait()
            pltpu.sync_copy(rows_v, out_hbm.at[pl.ds(base + off, _CHUNK)])
            return carry

        lax.fori_loop(0, _NCHUNK, body, 0)

    return _gather_rows


def kernel(input_ids, embed_table, fc_w, fc_b):
    m = _build_fused_table(embed_table, fc_w, fc_b)
    idx = input_ids.reshape(_NB)
    out = _make_gather_rows()(m, idx)
    return out.reshape(_BATCH, _SEQ, _VOCAB)


# speedup vs baseline: 1.0249x; 1.0249x over previous
import functools

import jax
import jax.numpy as jnp
from jax import lax
from jax.experimental import pallas as pl
from jax.experimental.pallas import tpu as pltpu
from jax.experimental.pallas import tpu_sc as plsc

_VOCAB = 1000
_HIDDEN = 16
_BATCH = 4096
_SEQ = 20
_NB = _BATCH * _SEQ

_NC = 2
_NS = 16
_NW = _NC * _NS
_B_PER_W = _NB // _NW
_CHUNK = 40
_NCHUNK = _B_PER_W // _CHUNK


def _fuse_table_kernel(e_ref, w_ref, b_ref, out_ref):
    m = lax.dot_general(
        e_ref[...], w_ref[...],
        dimension_numbers=(((1,), (1,)), ((), ())),
        preferred_element_type=jnp.float32,
    )
    out_ref[...] = m + b_ref[...]


def _build_fused_table(embed_table, fc_w, fc_b):
    return pl.pallas_call(
        _fuse_table_kernel,
        out_shape=jax.ShapeDtypeStruct((_VOCAB, _VOCAB), jnp.float32),
    )(embed_table, fc_w, fc_b.reshape(1, _VOCAB))


@functools.cache
def _make_gather_rows():
    mesh = plsc.VectorSubcoreMesh(core_axis_name="c", subcore_axis_name="s")

    @functools.partial(
        pl.kernel,
        mesh=mesh,
        out_type=jax.ShapeDtypeStruct((_NB, _VOCAB), jnp.float32),
        scratch_types=[
            pltpu.VMEM((_B_PER_W,), jnp.int32),
            pltpu.VMEM((_CHUNK, _VOCAB), jnp.float32),
            pltpu.VMEM((_CHUNK, _VOCAB), jnp.float32),
            pltpu.SemaphoreType.DMA,
            pltpu.SemaphoreType.DMA,
            pltpu.SemaphoreType.DMA,
            pltpu.SemaphoreType.DMA,
        ],
        compiler_params=pltpu.CompilerParams(use_tc_tiling_on_sc=False),
    )
    def _gather_rows(m_hbm, idx_hbm, out_hbm, idx_v, r0, r1, g0, g1, s0, s1):
        wid = lax.axis_index("s") * _NC + lax.axis_index("c")
        base = pl.multiple_of(wid * _B_PER_W, 8)
        pltpu.sync_copy(idx_hbm.at[pl.ds(base, _B_PER_W)], idx_v)

        def gth(c, buf, sem):
            off = pl.multiple_of(c * _CHUNK, 8)
            return pltpu.make_async_copy(
                m_hbm.at[idx_v.at[pl.ds(off, _CHUNK)]], buf, sem)

        def sct(c, buf, sem):
            off = pl.multiple_of(c * _CHUNK, 8)
            return pltpu.make_async_copy(
                buf, out_hbm.at[pl.ds(base + off, _CHUNK)], sem)

        gth(0, r0, g0).start()
        gth(1, r1, g1).start()
        npair = _NCHUNK // 2

        def body(i, carry):
            c = i * 2
            gth(c, r0, g0).wait()
            sct(c, r0, s0).start()
            gth(c + 1, r1, g1).wait()
            sct(c + 1, r1, s1).start()
            sct(c, r0, s0).wait()
            gth(c + 2, r0, g0).start()
            sct(c + 1, r1, s1).wait()
            gth(c + 3, r1, g1).start()
            return carry

        lax.fori_loop(0, npair - 1, body, 0)
        c_last = _NCHUNK - 2
        gth(c_last, r0, g0).wait()
        sct(c_last, r0, s0).start()
        gth(c_last + 1, r1, g1).wait()
        sct(c_last + 1, r1, s1).start()
        sct(c_last, r0, s0).wait()
        sct(c_last + 1, r1, s1).wait()

    return _gather_rows


def kernel(input_ids, embed_table, fc_w, fc_b):
    m = _build_fused_table(embed_table, fc_w, fc_b)
    idx = input_ids.reshape(_NB)
    out = _make_gather_rows()(m, idx)
    return out.reshape(_BATCH, _SEQ, _VOCAB)


# speedup vs baseline: 1.0285x; 1.0035x over previous
import functools

import jax
import jax.numpy as jnp
from jax import lax
from jax.experimental import pallas as pl
from jax.experimental.pallas import tpu as pltpu
from jax.experimental.pallas import tpu_sc as plsc

_VOCAB = 1000
_HIDDEN = 16
_BATCH = 4096
_SEQ = 20
_LPAD = 24

_NC = 2
_NS = 16
_NW = _NC * _NS
_SLABS_PER_W = _BATCH // _NW
_S = 2
_NCHUNK = _SLABS_PER_W // _S
_IDX_PER_W = _SLABS_PER_W * _LPAD


def _fuse_table_kernel(e_ref, w_ref, b_ref, out_ref):
    m = lax.dot_general(
        e_ref[...], w_ref[...],
        dimension_numbers=(((1,), (1,)), ((), ())),
        preferred_element_type=jnp.float32,
    )
    out_ref[...] = m + b_ref[...]


def _build_fused_table(embed_table, fc_w, fc_b):
    return pl.pallas_call(
        _fuse_table_kernel,
        out_shape=jax.ShapeDtypeStruct((_VOCAB, _VOCAB), jnp.float32),
    )(embed_table, fc_w, fc_b.reshape(1, _VOCAB))


@functools.cache
def _make_gather_rows():
    mesh = plsc.VectorSubcoreMesh(core_axis_name="c", subcore_axis_name="s")

    @functools.partial(
        pl.kernel,
        mesh=mesh,
        out_type=jax.ShapeDtypeStruct((_BATCH, _SEQ, _VOCAB), jnp.float32),
        scratch_types=[
            pltpu.VMEM((_IDX_PER_W,), jnp.int32),
            pltpu.VMEM((_S, _SEQ, _VOCAB), jnp.float32),
            pltpu.VMEM((_S, _SEQ, _VOCAB), jnp.float32),
            pltpu.SemaphoreType.DMA,
            pltpu.SemaphoreType.DMA,
            pltpu.SemaphoreType.DMA,
            pltpu.SemaphoreType.DMA,
        ],
        compiler_params=pltpu.CompilerParams(use_tc_tiling_on_sc=False),
    )
    def _gather_rows(m_hbm, idx_hbm, out_hbm, idx_v, a0, a1, g0, g1, s0, s1):
        wid = lax.axis_index("s") * _NC + lax.axis_index("c")
        slab0 = wid * _SLABS_PER_W
        ibase = pl.multiple_of(wid * _IDX_PER_W, 8)
        pltpu.sync_copy(idx_hbm.at[pl.ds(ibase, _IDX_PER_W)], idx_v)

        def gth(c, sb, buf, sem):
            off = pl.multiple_of((c * _S + sb) * _LPAD, 8)
            return pltpu.make_async_copy(
                m_hbm.at[idx_v.at[pl.ds(off, _SEQ)]], buf.at[sb], sem)

        def gth_start(c, buf, sem):
            for sb in range(_S):
                gth(c, sb, buf, sem).start()

        def gth_wait(c, buf, sem):
            for sb in range(_S):
                gth(c, sb, buf, sem).wait()

        def sct(c, buf, sem):
            return pltpu.make_async_copy(
                buf, out_hbm.at[pl.ds(slab0 + c * _S, _S)], sem)

        gth_start(0, a0, g0)
        gth_start(1, a1, g1)
        npair = _NCHUNK // 2

        def body(i, carry):
            c = i * 2
            gth_wait(c, a0, g0)
            sct(c, a0, s0).start()
            gth_wait(c + 1, a1, g1)
            sct(c + 1, a1, s1).start()
            sct(c, a0, s0).wait()
            gth_start(c + 2, a0, g0)
            sct(c + 1, a1, s1).wait()
            gth_start(c + 3, a1, g1)
            return carry

        lax.fori_loop(0, npair - 1, body, 0)
        c_last = _NCHUNK - 2
        gth_wait(c_last, a0, g0)
        sct(c_last, a0, s0).start()
        gth_wait(c_last + 1, a1, g1)
        sct(c_last + 1, a1, s1).start()
        sct(c_last, a0, s0).wait()
        sct(c_last + 1, a1, s1).wait()

    return _gather_rows


def kernel(input_ids, embed_table, fc_w, fc_b):
    m = _build_fused_table(embed_table, fc_w, fc_b)
    idx = jnp.pad(input_ids, ((0, 0), (0, _LPAD - _SEQ))).reshape(-1)
    return _make_gather_rows()(m, idx)


# speedup vs baseline: 5.0674x; 4.9271x over previous
import functools

import jax
import jax.numpy as jnp
from jax import lax
from jax.experimental import pallas as pl
from jax.experimental.pallas import tpu as pltpu
from jax.experimental.pallas import tpu_sc as plsc

_VOCAB = 1000
_HIDDEN = 16
_BATCH = 4096
_SEQ = 20

_NC = 2
_NS = 16
_NW = _NC * _NS
_BW = _BATCH // _NW
_BN = 1024


@functools.cache
def _make_hidden_t():
    mesh = plsc.VectorSubcoreMesh(core_axis_name="c", subcore_axis_name="s")

    @functools.partial(
        pl.kernel,
        mesh=mesh,
        out_type=jax.ShapeDtypeStruct((_SEQ, _HIDDEN, _BATCH), jnp.float32),
        scratch_types=[
            pltpu.VMEM((_VOCAB * _HIDDEN,), jnp.float32),
            pltpu.VMEM((_SEQ * _BW,), jnp.int32),
            pltpu.VMEM((_HIDDEN, _BW), jnp.float32),
        ],
        compiler_params=pltpu.CompilerParams(
            use_tc_tiling_on_sc=False, needs_layout_passes=False),
    )
    def _hidden_t(emb_hbm, ids_hbm, out_hbm, emb_v, ids_v, buf):
        wid = lax.axis_index("s") * _NC + lax.axis_index("c")
        b0 = pl.multiple_of(wid * _BW, 128)
        ibase = pl.multiple_of(wid * _SEQ * _BW, 8)
        pltpu.sync_copy(emb_hbm, emb_v)
        pltpu.sync_copy(ids_hbm.at[pl.ds(ibase, _SEQ * _BW)], ids_v)

        for l in range(_SEQ):
            def body(j, carry):
                off = pl.multiple_of(j * 16, 8)
                base = ids_v[pl.ds(l * _BW + off, 16)] * _HIDDEN
                for h in range(_HIDDEN):
                    buf[h, pl.ds(off, 16)] = plsc.load_gather(
                        emb_v, [base + h])
                return carry

            lax.fori_loop(0, _BW // 16, body, 0)
            pltpu.sync_copy(buf, out_hbm.at[l, :, pl.ds(b0, _BW)])

    return _hidden_t


def _logits_kernel(w_ref, b_ref, h_ref, out_ref):
    m = lax.dot_general(
        w_ref[...], h_ref[0],
        dimension_numbers=(((1,), (0,)), ((), ())),
        preferred_element_type=jnp.float32,
    )
    out_ref[0] = m + b_ref[...]


def _logits_t(fc_w, fc_b, hidden_t):
    return pl.pallas_call(
        _logits_kernel,
        grid=(_SEQ, _BATCH // _BN),
        in_specs=[
            pl.BlockSpec((_VOCAB, _HIDDEN), lambda l, n: (0, 0)),
            pl.BlockSpec((_VOCAB, 1), lambda l, n: (0, 0)),
            pl.BlockSpec((1, _HIDDEN, _BN), lambda l, n: (l, 0, n)),
        ],
        out_specs=pl.BlockSpec((1, _VOCAB, _BN), lambda l, n: (l, 0, n)),
        out_shape=jax.ShapeDtypeStruct((_SEQ, _VOCAB, _BATCH), jnp.float32),
    )(fc_w, fc_b.reshape(_VOCAB, 1), hidden_t)


def kernel(input_ids, embed_table, fc_w, fc_b):
    emb_flat = embed_table.reshape(_VOCAB * _HIDDEN)
    ids_w = (input_ids.T.reshape(_SEQ, _NW, _BW)
             .transpose(1, 0, 2).reshape(_SEQ * _BATCH))
    hidden_t = _make_hidden_t()(emb_flat, ids_w)
    out_t = _logits_t(fc_w, fc_b, hidden_t)
    return jnp.transpose(out_t, (2, 0, 1))


# speedup vs baseline: 5.1435x; 1.0150x over previous
import functools

import jax
import jax.numpy as jnp
from jax import lax
from jax.experimental import pallas as pl
from jax.experimental.pallas import tpu as pltpu
from jax.experimental.pallas import tpu_sc as plsc

_VOCAB = 1000
_HIDDEN = 16
_BATCH = 4096
_SEQ = 20

_NC = 2
_NS = 16
_NW = _NC * _NS
_BW = _BATCH // _NW
_BN = 1024


@functools.cache
def _make_hidden_t():
    mesh = plsc.VectorSubcoreMesh(core_axis_name="c", subcore_axis_name="s")

    @functools.partial(
        pl.kernel,
        mesh=mesh,
        out_type=jax.ShapeDtypeStruct((_SEQ, _HIDDEN, _BATCH), jnp.float32),
        scratch_types=[
            pltpu.VMEM((_VOCAB * _HIDDEN,), jnp.float32),
            pltpu.VMEM((_SEQ * _BW,), jnp.int32),
            pltpu.VMEM((_HIDDEN, _BW), jnp.float32),
            pltpu.VMEM((_HIDDEN, _BW), jnp.float32),
            pltpu.SemaphoreType.DMA,
            pltpu.SemaphoreType.DMA,
        ],
        compiler_params=pltpu.CompilerParams(
            use_tc_tiling_on_sc=False, needs_layout_passes=False),
    )
    def _hidden_t(emb_hbm, ids_hbm, out_hbm, emb_v, ids_v, b_0, b_1, s_0, s_1):
        wid = lax.axis_index("s") * _NC + lax.axis_index("c")
        b0 = pl.multiple_of(wid * _BW, 128)
        ibase = pl.multiple_of(wid * _SEQ * _BW, 8)
        pltpu.sync_copy(emb_hbm, emb_v)
        pltpu.sync_copy(ids_hbm.at[pl.ds(ibase, _SEQ * _BW)], ids_v)
        bufs = (b_0, b_1)
        sems = (s_0, s_1)

        def fill(l, buf):
            def body(j, carry):
                off = pl.multiple_of(j * 16, 8)
                base = ids_v[pl.ds(l * _BW + off, 16)] * _HIDDEN
                for h in range(_HIDDEN):
                    buf[h, pl.ds(off, 16)] = plsc.load_gather(
                        emb_v, [base + h])
                return carry

            lax.fori_loop(0, _BW // 16, body, 0)

        def flush(l, buf, sem):
            return pltpu.make_async_copy(
                buf, out_hbm.at[l, :, pl.ds(b0, _BW)], sem)

        for l in range(_SEQ):
            buf, sem = bufs[l % 2], sems[l % 2]
            if l >= 2:
                flush(l - 2, buf, sem).wait()
            fill(l, buf)
            flush(l, buf, sem).start()
        flush(_SEQ - 2, bufs[0], sems[0]).wait()
        flush(_SEQ - 1, bufs[1], sems[1]).wait()

    return _hidden_t


def _logits_kernel(w_ref, b_ref, h_ref, out_ref):
    m = lax.dot_general(
        w_ref[...], h_ref[0],
        dimension_numbers=(((1,), (0,)), ((), ())),
        preferred_element_type=jnp.float32,
    )
    out_ref[0] = m + b_ref[...]


def _logits_t(fc_w, fc_b, hidden_t):
    return pl.pallas_call(
        _logits_kernel,
        grid=(_SEQ, _BATCH // _BN),
        in_specs=[
            pl.BlockSpec((_VOCAB, _HIDDEN), lambda l, n: (0, 0)),
            pl.BlockSpec((_VOCAB, 1), lambda l, n: (0, 0)),
            pl.BlockSpec((1, _HIDDEN, _BN), lambda l, n: (l, 0, n)),
        ],
        out_specs=pl.BlockSpec((1, _VOCAB, _BN), lambda l, n: (l, 0, n)),
        out_shape=jax.ShapeDtypeStruct((_SEQ, _VOCAB, _BATCH), jnp.float32),
    )(fc_w, fc_b.reshape(_VOCAB, 1), hidden_t)


def kernel(input_ids, embed_table, fc_w, fc_b):
    emb_flat = embed_table.reshape(_VOCAB * _HIDDEN)
    ids_w = (input_ids.T.reshape(_SEQ, _NW, _BW)
             .transpose(1, 0, 2).reshape(_SEQ * _BATCH))
    hidden_t = _make_hidden_t()(emb_flat, ids_w)
    out_t = _logits_t(fc_w, fc_b, hidden_t)
    return jnp.transpose(out_t, (2, 0, 1))


# speedup vs baseline: 5.5554x; 1.0801x over previous
import functools

import jax
import jax.numpy as jnp
from jax import lax
from jax.experimental import pallas as pl
from jax.experimental.pallas import tpu as pltpu
from jax.experimental.pallas import tpu_sc as plsc

_VOCAB = 1000
_HIDDEN = 16
_BATCH = 4096
_SEQ = 20

_NC = 2
_NS = 16
_NW = _NC * _NS
_BW = _BATCH // _NW
_BN = 2048


@functools.cache
def _make_hidden_t():
    mesh = plsc.VectorSubcoreMesh(core_axis_name="c", subcore_axis_name="s")

    @functools.partial(
        pl.kernel,
        mesh=mesh,
        out_type=jax.ShapeDtypeStruct((_SEQ, _HIDDEN, _BATCH), jnp.float32),
        scratch_types=[
            pltpu.VMEM((_VOCAB * _HIDDEN,), jnp.float32),
            pltpu.VMEM((_SEQ * _BW,), jnp.int32),
            pltpu.VMEM((_HIDDEN, _BW), jnp.float32),
            pltpu.VMEM((_HIDDEN, _BW), jnp.float32),
            pltpu.SemaphoreType.DMA,
            pltpu.SemaphoreType.DMA,
        ],
        compiler_params=pltpu.CompilerParams(
            use_tc_tiling_on_sc=False, needs_layout_passes=False),
    )
    def _hidden_t(emb_hbm, ids_hbm, out_hbm, emb_v, ids_v, b_0, b_1, s_0, s_1):
        wid = lax.axis_index("s") * _NC + lax.axis_index("c")
        b0 = pl.multiple_of(wid * _BW, 128)
        ibase = pl.multiple_of(wid * _SEQ * _BW, 8)
        pltpu.sync_copy(emb_hbm, emb_v)
        pltpu.sync_copy(ids_hbm.at[pl.ds(ibase, _SEQ * _BW)], ids_v)
        bufs = (b_0, b_1)
        sems = (s_0, s_1)

        def fill(l, buf):
            def body(j, carry):
                off = pl.multiple_of(j * 16, 8)
                base = ids_v[pl.ds(l * _BW + off, 16)] * _HIDDEN
                for h in range(_HIDDEN):
                    buf[h, pl.ds(off, 16)] = plsc.load_gather(
                        emb_v, [base + h])
                return carry

            lax.fori_loop(0, _BW // 16, body, 0)

        def flush(l, buf, sem):
            return pltpu.make_async_copy(
                buf, out_hbm.at[l, :, pl.ds(b0, _BW)], sem)

        for l in range(_SEQ):
            buf, sem = bufs[l % 2], sems[l % 2]
            if l >= 2:
                flush(l - 2, buf, sem).wait()
            fill(l, buf)
            flush(l, buf, sem).start()
        flush(_SEQ - 2, bufs[0], sems[0]).wait()
        flush(_SEQ - 1, bufs[1], sems[1]).wait()

    return _hidden_t


def _logits_kernel(w_ref, b_ref, h_ref, out_ref):
    m = lax.dot_general(
        w_ref[...], h_ref[0],
        dimension_numbers=(((1,), (0,)), ((), ())),
        preferred_element_type=jnp.float32,
    )
    out_ref[0] = m + b_ref[...]


def _logits_t(fc_w, fc_b, hidden_t):
    return pl.pallas_call(
        _logits_kernel,
        grid=(_SEQ, _BATCH // _BN),
        in_specs=[
            pl.BlockSpec((_VOCAB, _HIDDEN), lambda l, n: (0, 0)),
            pl.BlockSpec((_VOCAB, 1), lambda l, n: (0, 0)),
            pl.BlockSpec((1, _HIDDEN, _BN), lambda l, n: (l, 0, n)),
        ],
        out_specs=pl.BlockSpec((1, _VOCAB, _BN), lambda l, n: (l, 0, n)),
        out_shape=jax.ShapeDtypeStruct((_SEQ, _VOCAB, _BATCH), jnp.float32),
    )(fc_w, fc_b.reshape(_VOCAB, 1), hidden_t)


def kernel(input_ids, embed_table, fc_w, fc_b):
    emb_flat = embed_table.reshape(_VOCAB * _HIDDEN)
    ids_w = (input_ids.T.reshape(_SEQ, _NW, _BW)
             .transpose(1, 0, 2).reshape(_SEQ * _BATCH))
    hidden_t = _make_hidden_t()(emb_flat, ids_w)
    out_t = _logits_t(fc_w, fc_b, hidden_t)
    return jnp.transpose(out_t, (2, 0, 1))
